# Initial kernel scaffold; baseline (speedup 1.0000x reference)
import jax
import jax.numpy as jnp
from jax.experimental import pallas as pl


def kernel(x, adj_t, adj_t2):
    raise NotImplementedError("write your pallas kernel here")


# speedup vs baseline: 1.0264x; 1.0264x over previous
import jax
import jax.numpy as jnp
from jax.experimental import pallas as pl

N = 10000
D = 128
BM = 200


def _gcn_block_kernel(x_ref, a1_ref, a2_ref, out_ref):
    xb = x_ref[...]
    a1 = a1_ref[...].astype(jnp.bfloat16)
    a2 = a2_ref[...].astype(jnp.bfloat16)
    out_ref[:, :D] = jnp.dot(a1, xb, preferred_element_type=jnp.float32)
    out_ref[:, D:] = jnp.dot(a2, xb, preferred_element_type=jnp.float32)


def kernel(x, adj_t, adj_t2):
    n, d = x.shape
    bm = BM if n % BM == 0 else n
    x_bf = x.astype(jnp.bfloat16)
    return pl.pallas_call(
        _gcn_block_kernel,
        grid=(n // bm,),
        in_specs=[
            pl.BlockSpec((n, d), lambda i: (0, 0)),
            pl.BlockSpec((bm, n), lambda i: (i, 0)),
            pl.BlockSpec((bm, n), lambda i: (i, 0)),
        ],
        out_specs=pl.BlockSpec((bm, 2 * d), lambda i: (i, 0)),
        out_shape=jax.ShapeDtypeStruct((n, 2 * d), jnp.float32),
    )(x_bf, adj_t, adj_t2)


# speedup vs baseline: 1.0359x; 1.0092x over previous
import jax
import jax.numpy as jnp
from jax.experimental import pallas as pl

N = 10000
D = 128
BM = 200


def _gcn_block_kernel(x_ref, a1_ref, a2_ref, out_ref):
    xb = x_ref[...].astype(jnp.bfloat16)
    a1 = a1_ref[...].astype(jnp.bfloat16)
    a2 = a2_ref[...].astype(jnp.bfloat16)
    out_ref[:, :D] = jnp.dot(a1, xb, preferred_element_type=jnp.float32)
    out_ref[:, D:] = jnp.dot(a2, xb, preferred_element_type=jnp.float32)


def kernel(x, adj_t, adj_t2):
    n, d = x.shape
    bm = BM if n % BM == 0 else n
    return pl.pallas_call(
        _gcn_block_kernel,
        grid=(n // bm,),
        in_specs=[
            pl.BlockSpec((n, d), lambda i: (0, 0)),
            pl.BlockSpec((bm, n), lambda i: (i, 0)),
            pl.BlockSpec((bm, n), lambda i: (i, 0)),
        ],
        out_specs=pl.BlockSpec((bm, 2 * d), lambda i: (i, 0)),
        out_shape=jax.ShapeDtypeStruct((n, 2 * d), jnp.float32),
    )(x, adj_t, adj_t2)
